# baseline (speedup 1.0000x reference)
import functools

import jax
import jax.numpy as jnp
from jax import lax
from jax.experimental import pallas as pl
from jax.experimental.pallas import tpu as pltpu
from jax.experimental.pallas import tpu_sc as plsc

N_SC_CALLS = 16
CCOLS = 32
K_NNZ = 128
N_TILES = 16
MM_COLS = 256
PIECES = MM_COLS // CCOLS


def _mm_body(x_ref, w_ref, *out_refs):
    y = jnp.dot(x_ref[...], w_ref[...], preferred_element_type=jnp.float32)
    for j, o_ref in enumerate(out_refs):
        o_ref[...] = y[:, j * CCOLS:(j + 1) * CCOLS]


def _matmul_chunk(x, W, i):
    n, d = x.shape
    rb = 2048
    return pl.pallas_call(
        _mm_body,
        grid=(n // rb,),
        in_specs=[
            pl.BlockSpec((rb, d), lambda m: (m, 0)),
            pl.BlockSpec((d, MM_COLS), lambda m, i=i: (0, i)),
        ],
        out_specs=[pl.BlockSpec((rb, CCOLS), lambda m: (m, 0))] * PIECES,
        out_shape=[jax.ShapeDtypeStruct((n, CCOLS), jnp.float32)] * PIECES,
    )(x, W)


def _sc_spmm(ylo, yhi, col2, row2, vrep3, chunks_per_tile):
    n = ylo.shape[0]
    mesh = plsc.VectorSubcoreMesh(core_axis_name="c", subcore_axis_name="s")

    @functools.partial(
        pl.kernel,
        out_type=jax.ShapeDtypeStruct((2, n, CCOLS), jnp.float32),
        mesh=mesh,
        scratch_types=[
            pltpu.VMEM((chunks_per_tile, K_NNZ), jnp.int32),
            pltpu.VMEM((chunks_per_tile, K_NNZ), jnp.int32),
            pltpu.VMEM((4, K_NNZ, CCOLS), jnp.float32),
            pltpu.VMEM((4, K_NNZ, 16), jnp.float32),
            pltpu.VMEM_SHARED((n, CCOLS), jnp.float32),
            pltpu.SemaphoreType.DMA((4,)),
            pltpu.SemaphoreType.DMA((4,)),
            pltpu.SemaphoreType.DMA((4,)),
        ],
        compiler_params=pltpu.CompilerParams(use_tc_tiling_on_sc=False),
    )
    def k(ylo_hbm, yhi_hbm, col_hbm, row_hbm, vrep_hbm, out_hbm,
          colv, rowv, gbuf, vbuf, acc, gsem, vsem, ssem):
        c = lax.axis_index("c")
        s = lax.axis_index("s")
        rows_per_tile = n // N_TILES
        nchunk = chunks_per_tile

        @pl.loop(0, K_NNZ)
        def _(kk):
            for j in range(CCOLS // 16):
                gbuf[0, kk, pl.ds(16 * j, 16)] = jnp.zeros((16,), jnp.float32)

        for i in range(rows_per_tile // K_NNZ):
            pltpu.sync_copy(
                gbuf.at[0], acc.at[pl.ds(s * rows_per_tile + i * K_NNZ, K_NNZ)])
        plsc.subcore_barrier()

        pltpu.sync_copy(col_hbm.at[pl.ds(s * nchunk, nchunk)], colv)
        pltpu.sync_copy(row_hbm.at[pl.ds(s * nchunk, nchunk)], rowv)

        def main_loop(y_hbm):
            def fetch(gg, b):
                pltpu.make_async_copy(
                    vrep_hbm.at[s * nchunk + gg], vbuf.at[b], vsem.at[b]
                ).start()
                pltpu.make_async_copy(
                    y_hbm.at[colv.at[gg]], gbuf.at[b], gsem.at[b]
                ).start()

            fetch(0, 0)
            fetch(1, 1)

            @pl.loop(0, nchunk, step=4)
            def _(g):
                for b in range(4):
                    gg = g + b
                    pltpu.make_async_copy(
                        vrep_hbm.at[s * nchunk + gg], vbuf.at[b], vsem.at[b]
                    ).wait()
                    pltpu.make_async_copy(
                        y_hbm.at[colv.at[gg]], gbuf.at[b], gsem.at[b]
                    ).wait()

                    @pl.loop(0, K_NNZ, step=16)
                    def _(k0):
                        for kd in range(16):
                            kk = k0 + kd
                            vv = vbuf[b, kk, pl.ds(0, 16)]
                            for j in range(CCOLS // 16):
                                sl = pl.ds(16 * j, 16)
                                gbuf[b, kk, sl] = gbuf[b, kk, sl] * vv

                    pltpu.async_copy(
                        gbuf.at[b], acc.at[rowv.at[gg]], ssem.at[b], add=True)

                    b2 = (b + 2) % 4

                    @pl.when(gg + 2 < nchunk)
                    def _():
                        @pl.when(gg >= 2)
                        def _():
                            pltpu.make_async_copy(
                                y_hbm.at[pl.ds(0, K_NNZ)], gbuf.at[b2],
                                ssem.at[b2]).wait()
                        fetch(gg + 2, b2)

            for b in range(4):
                pltpu.make_async_copy(
                    y_hbm.at[pl.ds(0, K_NNZ)], gbuf.at[b], ssem.at[b]).wait()

        @pl.when(c == 0)
        def _():
            main_loop(ylo_hbm)

        @pl.when(c == 1)
        def _():
            main_loop(yhi_hbm)

        plsc.subcore_barrier()
        pltpu.sync_copy(acc.at[pl.ds(s * rows_per_tile, rows_per_tile)],
                        out_hbm.at[c, pl.ds(s * rows_per_tile, rows_per_tile)])

    return k(ylo, yhi, col2, row2, vrep3)


def _post_body(b_ref, *refs):
    zrefs, out_ref = refs[:-1], refs[-1]
    pieces = []
    for zr in zrefs:
        z = zr[...]
        pieces.append(z[0])
        pieces.append(z[1])
    h = jnp.concatenate(pieces, axis=-1) + b_ref[...]
    h = h * 0.5 * (1.0 + lax.erf(h * (2.0 ** -0.5)))
    denom = jnp.maximum(jnp.sqrt(jnp.sum(h * h, axis=-1, keepdims=True)), 1e-8)
    out_ref[...] = h / denom


def _postprocess(z_list, b):
    n = z_list[0].shape[1]
    d = b.shape[0]
    rb = 512
    b2 = b.reshape(1, d)
    return pl.pallas_call(
        _post_body,
        grid=(n // rb,),
        in_specs=[pl.BlockSpec((1, d), lambda m: (0, 0))]
        + [pl.BlockSpec((2, rb, CCOLS), lambda m: (0, m, 0))] * len(z_list),
        out_specs=pl.BlockSpec((rb, d), lambda m: (m, 0)),
        out_shape=jax.ShapeDtypeStruct((n, d), jnp.float32),
    )(b2, *z_list)


def kernel(x, mapping_indices, mapping_values, W, b):
    n, d = x.shape
    nnz = mapping_indices.shape[1]
    row = mapping_indices[0]
    col = mapping_indices[1]
    val = mapping_values

    per_tile = -(-nnz // (N_TILES * K_NNZ * 8)) * K_NNZ * 8
    p = per_tile * N_TILES
    chunks_per_tile = per_tile // K_NNZ
    row = jnp.pad(row, (0, p - nnz))
    col = jnp.pad(col, (0, p - nnz))
    val = jnp.pad(val, (0, p - nnz))

    col2 = col.reshape(p // K_NNZ, K_NNZ)
    row2 = row.reshape(p // K_NNZ, K_NNZ)
    vrep3 = jnp.broadcast_to(val[:, None], (p, 16)).reshape(p // K_NNZ, K_NNZ, 16)

    z_list = []
    for i in range(d // MM_COLS):
        pieces = _matmul_chunk(x, W, i)
        for j in range(PIECES // 2):
            z_list.append(_sc_spmm(pieces[2 * j], pieces[2 * j + 1],
                                   col2, row2, vrep3, chunks_per_tile))
    return _postprocess(z_list, b)


# speedup vs baseline: 1.1392x; 1.1392x over previous
import functools

import jax
import jax.numpy as jnp
from jax import lax
from jax.experimental import pallas as pl
from jax.experimental.pallas import tpu as pltpu
from jax.experimental.pallas import tpu_sc as plsc

N_SC_CALLS = 8
CCOLS = 64
K_NNZ = 128
N_TILES = 16
MM_COLS = 256
PIECES = MM_COLS // CCOLS


def _mm_body(x_ref, w_ref, *out_refs):
    y = jnp.dot(x_ref[...], w_ref[...], preferred_element_type=jnp.float32)
    for j, o_ref in enumerate(out_refs):
        o_ref[...] = y[:, j * CCOLS:(j + 1) * CCOLS]


def _matmul_chunk(x, W, i):
    n, d = x.shape
    rb = 2048
    return pl.pallas_call(
        _mm_body,
        grid=(n // rb,),
        in_specs=[
            pl.BlockSpec((rb, d), lambda m: (m, 0)),
            pl.BlockSpec((d, MM_COLS), lambda m, i=i: (0, i)),
        ],
        out_specs=[pl.BlockSpec((rb, CCOLS), lambda m: (m, 0))] * PIECES,
        out_shape=[jax.ShapeDtypeStruct((n, CCOLS), jnp.float32)] * PIECES,
    )(x, W)


def _sc_spmm(ylo, yhi, col2, row2, vrep3, chunks_per_tile):
    n = ylo.shape[0]
    mesh = plsc.VectorSubcoreMesh(core_axis_name="c", subcore_axis_name="s")

    @functools.partial(
        pl.kernel,
        out_type=jax.ShapeDtypeStruct((2, n, CCOLS), jnp.float32),
        mesh=mesh,
        scratch_types=[
            pltpu.VMEM((chunks_per_tile, K_NNZ), jnp.int32),
            pltpu.VMEM((chunks_per_tile, K_NNZ), jnp.int32),
            pltpu.VMEM((2, K_NNZ, CCOLS), jnp.float32),
            pltpu.VMEM((2, K_NNZ, 16), jnp.float32),
            pltpu.VMEM_SHARED((n, CCOLS), jnp.float32),
            pltpu.SemaphoreType.DMA((2,)),
            pltpu.SemaphoreType.DMA((2,)),
        ],
        compiler_params=pltpu.CompilerParams(use_tc_tiling_on_sc=False),
    )
    def k(ylo_hbm, yhi_hbm, col_hbm, row_hbm, vrep_hbm, out_hbm,
          colv, rowv, gbuf, vbuf, acc, gsem, vsem):
        c = lax.axis_index("c")
        s = lax.axis_index("s")
        rows_per_tile = n // N_TILES
        nchunk = chunks_per_tile

        @pl.loop(0, K_NNZ)
        def _(kk):
            for j in range(CCOLS // 16):
                gbuf[0, kk, pl.ds(16 * j, 16)] = jnp.zeros((16,), jnp.float32)

        for i in range(rows_per_tile // K_NNZ):
            pltpu.sync_copy(
                gbuf.at[0], acc.at[pl.ds(s * rows_per_tile + i * K_NNZ, K_NNZ)])
        plsc.subcore_barrier()

        pltpu.sync_copy(col_hbm.at[pl.ds(s * nchunk, nchunk)], colv)
        pltpu.sync_copy(row_hbm.at[pl.ds(s * nchunk, nchunk)], rowv)

        def main_loop(y_hbm):
            def fetch(gg, b):
                pltpu.make_async_copy(
                    vrep_hbm.at[s * nchunk + gg], vbuf.at[b], vsem.at[b]
                ).start()
                pltpu.make_async_copy(
                    y_hbm.at[colv.at[gg]], gbuf.at[b], gsem.at[b]
                ).start()

            fetch(0, 0)
            fetch(1, 1)

            @pl.loop(0, nchunk, step=2)
            def _(g):
                for b in range(2):
                    gg = g + b
                    pltpu.make_async_copy(
                        vrep_hbm.at[s * nchunk + gg], vbuf.at[b], vsem.at[b]
                    ).wait()
                    pltpu.make_async_copy(
                        y_hbm.at[colv.at[gg]], gbuf.at[b], gsem.at[b]
                    ).wait()

                    @pl.loop(0, K_NNZ, step=1, unroll=16)
                    def _(kk):
                        vv = vbuf[b, kk, pl.ds(0, 16)]
                        for j in range(CCOLS // 16):
                            sl = pl.ds(16 * j, 16)
                            gbuf[b, kk, sl] = gbuf[b, kk, sl] * vv

                    pltpu.sync_copy(gbuf.at[b], acc.at[rowv.at[gg]], add=True)

                    @pl.when(gg + 2 < nchunk)
                    def _():
                        fetch(gg + 2, b)

        @pl.when(c == 0)
        def _():
            main_loop(ylo_hbm)

        @pl.when(c == 1)
        def _():
            main_loop(yhi_hbm)

        plsc.subcore_barrier()
        pltpu.sync_copy(acc.at[pl.ds(s * rows_per_tile, rows_per_tile)],
                        out_hbm.at[c, pl.ds(s * rows_per_tile, rows_per_tile)])

    return k(ylo, yhi, col2, row2, vrep3)


def _post_body(b_ref, *refs):
    zrefs, out_ref = refs[:-1], refs[-1]
    pieces = []
    for zr in zrefs:
        z = zr[...]
        pieces.append(z[0])
        pieces.append(z[1])
    h = jnp.concatenate(pieces, axis=-1) + b_ref[...]
    h = h * 0.5 * (1.0 + lax.erf(h * (2.0 ** -0.5)))
    denom = jnp.maximum(jnp.sqrt(jnp.sum(h * h, axis=-1, keepdims=True)), 1e-8)
    out_ref[...] = h / denom


def _postprocess(z_list, b):
    n = z_list[0].shape[1]
    d = b.shape[0]
    rb = 512
    b2 = b.reshape(1, d)
    return pl.pallas_call(
        _post_body,
        grid=(n // rb,),
        in_specs=[pl.BlockSpec((1, d), lambda m: (0, 0))]
        + [pl.BlockSpec((2, rb, CCOLS), lambda m: (0, m, 0))] * len(z_list),
        out_specs=pl.BlockSpec((rb, d), lambda m: (m, 0)),
        out_shape=jax.ShapeDtypeStruct((n, d), jnp.float32),
    )(b2, *z_list)


def kernel(x, mapping_indices, mapping_values, W, b):
    n, d = x.shape
    nnz = mapping_indices.shape[1]
    row = mapping_indices[0]
    col = mapping_indices[1]
    val = mapping_values

    per_tile = -(-nnz // (N_TILES * K_NNZ * 8)) * K_NNZ * 8
    p = per_tile * N_TILES
    chunks_per_tile = per_tile // K_NNZ
    row = jnp.pad(row, (0, p - nnz))
    col = jnp.pad(col, (0, p - nnz))
    val = jnp.pad(val, (0, p - nnz))

    col2 = col.reshape(p // K_NNZ, K_NNZ)
    row2 = row.reshape(p // K_NNZ, K_NNZ)
    vrep3 = jnp.broadcast_to(val[:, None], (p, 16)).reshape(p // K_NNZ, K_NNZ, 16)

    z_list = []
    for i in range(d // MM_COLS):
        pieces = _matmul_chunk(x, W, i)
        for j in range(PIECES // 2):
            z_list.append(_sc_spmm(pieces[2 * j], pieces[2 * j + 1],
                                   col2, row2, vrep3, chunks_per_tile))
    return _postprocess(z_list, b)
